# baseline (speedup 1.0000x reference)
import jax
import jax.numpy as jnp
from jax.experimental import pallas as pl

_STEP_CURRENT = 10
_N_HIST = _STEP_CURRENT + 1
_N_TARGET = 8
_PI = 3.141592653589793
_HI = jax.lax.Precision.HIGHEST


def _wrap_rad(x):
    m = x + _PI
    m = m - (2.0 * _PI) * jnp.floor(m / (2.0 * _PI))
    return m - _PI


def _dot_t(a, b):
    return jax.lax.dot_general(
        a, b, (((1,), (1,)), ((), ())), precision=_HI,
        preferred_element_type=jnp.float32)


def _dot(a, b):
    return jax.lax.dot_general(
        a, b, (((1,), (0,)), ((), ())), precision=_HI,
        preferred_element_type=jnp.float32)


def _scene_kernel(valid_ref, pos_ref, vel_ref, spd_ref, acc_ref, yaw_ref,
                  yawr_ref, type_ref, role_ref, size_ref, cmd_ref,
                  o_idx, o_refpos, o_refrot, o_type, o_role,
                  o_tvalid, o_tpos, o_tvel, o_tspd, o_tacc, o_tyaw, o_tyawr,
                  o_size, o_gvalid, o_gpos, o_gspd, o_gvel, o_gyaw, o_gcmd):
    A = 64
    P = _N_TARGET
    T = valid_ref.shape[1]

    valid = valid_ref[0].astype(jnp.float32)
    role = role_ref[0].astype(jnp.float32)

    w_col = jnp.sum(role, axis=1, keepdims=True)
    w_row = jnp.transpose(w_col) + valid[_STEP_CURRENT:_STEP_CURRENT + 1, :]
    key_row = w_row.astype(jnp.int32) * A + (A - 1 - jax.lax.broadcasted_iota(
        jnp.int32, (1, A), 1))
    key_col = jnp.transpose(key_row)
    rank_col = jnp.sum((key_row > key_col).astype(jnp.int32), axis=1,
                       keepdims=True)
    rank_row = jnp.transpose(rank_col)
    p_col = jax.lax.broadcasted_iota(jnp.int32, (P, 1), 0)
    sel = (rank_row == p_col)
    a_row = jax.lax.broadcasted_iota(jnp.int32, (P, A), 1)
    idx_col = jnp.sum(jnp.where(sel, a_row, 0), axis=1, keepdims=True)
    o_idx[0] = jnp.transpose(idx_col)

    sel_f = sel.astype(jnp.float32)

    l_row = jax.lax.broadcasted_iota(jnp.int32, (A, 2 * A), 1)
    a2_col = 2 * jax.lax.broadcasted_iota(jnp.int32, (A, 2 * A), 0)
    d0 = (l_row == a2_col).astype(jnp.float32)
    d1 = (l_row == a2_col + 1).astype(jnp.float32)
    sel2x = _dot(sel_f, d0)
    sel2y = _dot(sel_f, d1)

    pos = pos_ref[0]
    vel = vel_ref[0]
    px = _dot_t(sel2x, pos)
    py = _dot_t(sel2y, pos)
    vx = _dot_t(sel2x, vel)
    vy = _dot_t(sel2y, vel)
    g_spd = _dot_t(sel_f, spd_ref[0])
    g_acc = _dot_t(sel_f, acc_ref[0])
    g_yaw = _dot_t(sel_f, yaw_ref[0])
    g_yawr = _dot_t(sel_f, yawr_ref[0])
    g_valid = _dot_t(sel_f, valid)

    g_type = _dot(sel_f, type_ref[0].astype(jnp.float32))
    g_role = _dot(sel_f, role)
    g_size = _dot(sel_f, size_ref[0])
    g_cmd = _dot(sel_f, cmd_ref[0])

    px0 = px[:, _STEP_CURRENT:_STEP_CURRENT + 1]
    py0 = py[:, _STEP_CURRENT:_STEP_CURRENT + 1]
    yaw0 = g_yaw[:, _STEP_CURRENT:_STEP_CURRENT + 1]
    c = jnp.cos(yaw0)
    s = jnp.sin(yaw0)

    dx = px - px0
    dy = py - py0
    lx = dx * c + dy * s
    ly = dy * c - dx * s
    lvx = vx * c + vy * s
    lvy = vy * c - vx * s
    lyaw = _wrap_rad(g_yaw - yaw0)

    o_refpos[0] = jnp.concatenate([px0, py0], axis=1)
    o_refrot[0] = jnp.concatenate([c, -s, s, c], axis=1)
    o_type[0] = g_type > 0.5
    o_role[0] = g_role > 0.5
    o_size[0] = g_size
    o_gcmd[0] = g_cmd

    H = _N_HIST
    t_row_h = jax.lax.broadcasted_iota(jnp.int32, (H, 2 * H), 1)
    t2_col_h = 2 * jax.lax.broadcasted_iota(jnp.int32, (H, 2 * H), 0)
    e0h = (t_row_h == t2_col_h).astype(jnp.float32)
    e1h = (t_row_h == t2_col_h + 1).astype(jnp.float32)
    F = T - H
    t_row_f = jax.lax.broadcasted_iota(jnp.int32, (F, 2 * F), 1)
    t2_col_f = 2 * jax.lax.broadcasted_iota(jnp.int32, (F, 2 * F), 0)
    e0f = (t_row_f == t2_col_f).astype(jnp.float32)
    e1f = (t_row_f == t2_col_f + 1).astype(jnp.float32)

    o_tvalid[0] = g_valid[:, :H] > 0.5
    o_tpos[0] = _dot(lx[:, :H], e0h) + _dot(ly[:, :H], e1h)
    o_tvel[0] = _dot(lvx[:, :H], e0h) + _dot(lvy[:, :H], e1h)
    o_tspd[0] = g_spd[:, :H]
    o_tacc[0] = g_acc[:, :H]
    o_tyaw[0] = lyaw[:, :H]
    o_tyawr[0] = g_yawr[:, :H]

    o_gvalid[0] = g_valid[:, H:] > 0.5
    o_gpos[0] = _dot(lx[:, H:], e0f) + _dot(ly[:, H:], e1f)
    o_gvel[0] = _dot(lvx[:, H:], e0f) + _dot(lvy[:, H:], e1f)
    o_gspd[0] = g_spd[:, H:]
    o_gyaw[0] = lyaw[:, H:]


def kernel(agent_valid, agent_pos, agent_vel, agent_spd, agent_acc,
           agent_yaw_bbox, agent_yaw_rate, agent_type, agent_role,
           agent_size, agent_cmd):
    S, T, A = agent_valid.shape
    P = _N_TARGET
    H = _N_HIST
    F = T - H

    pos = agent_pos.reshape(S, T, 2 * A)
    vel = agent_vel.reshape(S, T, 2 * A)
    spd = agent_spd.reshape(S, T, A)
    acc = agent_acc.reshape(S, T, A)
    yaw = agent_yaw_bbox.reshape(S, T, A)
    yawr = agent_yaw_rate.reshape(S, T, A)

    f32 = jnp.float32
    out_shapes = (
        jax.ShapeDtypeStruct((S, 1, P), jnp.int32),
        jax.ShapeDtypeStruct((S, P, 2), f32),
        jax.ShapeDtypeStruct((S, P, 4), f32),
        jax.ShapeDtypeStruct((S, P, 3), jnp.bool_),
        jax.ShapeDtypeStruct((S, P, 3), jnp.bool_),
        jax.ShapeDtypeStruct((S, P, H), jnp.bool_),
        jax.ShapeDtypeStruct((S, P, 2 * H), f32),
        jax.ShapeDtypeStruct((S, P, 2 * H), f32),
        jax.ShapeDtypeStruct((S, P, H), f32),
        jax.ShapeDtypeStruct((S, P, H), f32),
        jax.ShapeDtypeStruct((S, P, H), f32),
        jax.ShapeDtypeStruct((S, P, H), f32),
        jax.ShapeDtypeStruct((S, P, 3), f32),
        jax.ShapeDtypeStruct((S, P, F), jnp.bool_),
        jax.ShapeDtypeStruct((S, P, 2 * F), f32),
        jax.ShapeDtypeStruct((S, P, F), f32),
        jax.ShapeDtypeStruct((S, P, 2 * F), f32),
        jax.ShapeDtypeStruct((S, P, F), f32),
        jax.ShapeDtypeStruct((S, P, 8), f32),
    )

    def spec(*dims):
        return pl.BlockSpec((1,) + dims, lambda s: (s,) + (0,) * len(dims))

    outs = pl.pallas_call(
        _scene_kernel,
        grid=(S,),
        in_specs=[
            spec(T, A),
            spec(T, 2 * A),
            spec(T, 2 * A),
            spec(T, A),
            spec(T, A),
            spec(T, A),
            spec(T, A),
            spec(A, 3),
            spec(A, 3),
            spec(A, 3),
            spec(A, 8),
        ],
        out_specs=tuple(spec(*o.shape[1:]) for o in out_shapes),
        out_shape=out_shapes,
    )(agent_valid, pos, vel, spd, acc, yaw, yawr,
      agent_type, agent_role, agent_size, agent_cmd)

    (o_idx, o_refpos, o_refrot, o_type, o_role, o_tvalid, o_tpos, o_tvel,
     o_tspd, o_tacc, o_tyaw, o_tyawr, o_size, o_gvalid, o_gpos, o_gspd,
     o_gvel, o_gyaw, o_gcmd) = outs

    target_indices = o_idx.reshape(S, P)
    ref_pos = o_refpos.reshape(S, P, 1, 2)
    ref_rot = o_refrot.reshape(S, P, 2, 2)
    tgt_pos = o_tpos.reshape(S, P, H, 2)
    tgt_vel = o_tvel.reshape(S, P, H, 2)
    gt_pos = o_gpos.reshape(S, P, F, 2)
    gt_vel = o_gvel.reshape(S, P, F, 2)

    return (target_indices, ref_pos, ref_rot, o_type, o_role,
            o_tvalid, tgt_pos, tgt_vel,
            o_tspd.reshape(S, P, H, 1), o_tacc.reshape(S, P, H, 1),
            o_tyaw.reshape(S, P, H, 1), o_tyawr.reshape(S, P, H, 1),
            o_type, o_role, o_size,
            o_gvalid, gt_pos, o_gspd.reshape(S, P, F, 1), gt_vel,
            o_gyaw.reshape(S, P, F, 1), o_gcmd)


# speedup vs baseline: 1.1284x; 1.1284x over previous
import jax
import jax.numpy as jnp
from jax.experimental import pallas as pl
from jax.experimental.pallas import tpu as pltpu

_STEP_CURRENT = 10
_N_HIST = _STEP_CURRENT + 1
_N_TARGET = 8
_PI = 3.141592653589793
_HI = jax.lax.Precision.HIGHEST
_LO = jax.lax.Precision.DEFAULT
_SCENES_PER_STEP = 4


def _wrap_rad(x):
    m = x + _PI
    m = m - (2.0 * _PI) * jnp.floor(m / (2.0 * _PI))
    return m - _PI


def _dot_t(a, b, prec=_HI):
    return jax.lax.dot_general(
        a, b, (((1,), (1,)), ((), ())), precision=prec,
        preferred_element_type=jnp.float32)


def _dot(a, b, prec=_HI):
    return jax.lax.dot_general(
        a, b, (((1,), (0,)), ((), ())), precision=prec,
        preferred_element_type=jnp.float32)


def _group_kernel(valid_ref, pos_ref, vel_ref, spd_ref, acc_ref, yaw_ref,
                  yawr_ref, type_ref, role_ref, size_ref, cmd_ref, *out_refs):
    for g in range(_SCENES_PER_STEP):
        _one_scene(g, valid_ref, pos_ref, vel_ref, spd_ref, acc_ref, yaw_ref,
                   yawr_ref, type_ref, role_ref, size_ref, cmd_ref, *out_refs)


def _one_scene(g, valid_ref, pos_ref, vel_ref, spd_ref, acc_ref, yaw_ref,
               yawr_ref, type_ref, role_ref, size_ref, cmd_ref,
               o_idx, o_refpos, o_refrot, o_type, o_role,
               o_tvalid, o_tpos, o_tvel, o_tspd, o_tacc, o_tyaw, o_tyawr,
               o_size, o_gvalid, o_gpos, o_gspd, o_gvel, o_gyaw, o_gcmd):
    A = 64
    P = _N_TARGET
    T = valid_ref.shape[1]

    valid = valid_ref[g].astype(jnp.float32)
    role = role_ref[g].astype(jnp.float32)

    w_col = jnp.sum(role, axis=1, keepdims=True)
    w_row = jnp.transpose(w_col) + valid[_STEP_CURRENT:_STEP_CURRENT + 1, :]
    key_row = w_row.astype(jnp.int32) * A + (A - 1 - jax.lax.broadcasted_iota(
        jnp.int32, (1, A), 1))
    key_col = jnp.transpose(key_row)
    rank_col = jnp.sum((key_row > key_col).astype(jnp.int32), axis=1,
                       keepdims=True)
    rank_row = jnp.transpose(rank_col)
    p_col = jax.lax.broadcasted_iota(jnp.int32, (P, 1), 0)
    sel = (rank_row == p_col)
    a_row = jax.lax.broadcasted_iota(jnp.int32, (P, A), 1)
    idx_col = jnp.sum(jnp.where(sel, a_row, 0), axis=1, keepdims=True)
    o_idx[g] = jnp.transpose(idx_col)

    sel_f = sel.astype(jnp.float32)

    l_row = jax.lax.broadcasted_iota(jnp.int32, (A, 2 * A), 1)
    a2_col = 2 * jax.lax.broadcasted_iota(jnp.int32, (A, 2 * A), 0)
    d0 = (l_row == a2_col).astype(jnp.float32)
    d1 = (l_row == a2_col + 1).astype(jnp.float32)
    sel2x = _dot(sel_f, d0, _LO)
    sel2y = _dot(sel_f, d1, _LO)

    pos = pos_ref[g]
    vel = vel_ref[g]
    px = _dot_t(sel2x, pos)
    py = _dot_t(sel2y, pos)
    vx = _dot_t(sel2x, vel)
    vy = _dot_t(sel2y, vel)
    g_spd = _dot_t(sel_f, spd_ref[g])
    g_acc = _dot_t(sel_f, acc_ref[g])
    g_yaw = _dot_t(sel_f, yaw_ref[g])
    g_yawr = _dot_t(sel_f, yawr_ref[g])
    g_valid = _dot_t(sel_f, valid)

    g_type = _dot(sel_f, type_ref[g].astype(jnp.float32))
    g_role = _dot(sel_f, role)
    g_size = _dot(sel_f, size_ref[g])
    g_cmd = _dot(sel_f, cmd_ref[g])

    px0 = px[:, _STEP_CURRENT:_STEP_CURRENT + 1]
    py0 = py[:, _STEP_CURRENT:_STEP_CURRENT + 1]
    yaw0 = g_yaw[:, _STEP_CURRENT:_STEP_CURRENT + 1]
    c = jnp.cos(yaw0)
    s = jnp.sin(yaw0)

    dx = px - px0
    dy = py - py0
    lx = dx * c + dy * s
    ly = dy * c - dx * s
    lvx = vx * c + vy * s
    lvy = vy * c - vx * s
    lyaw = _wrap_rad(g_yaw - yaw0)

    o_refpos[g] = jnp.concatenate([px0, py0], axis=1)
    o_refrot[g] = jnp.concatenate([c, -s, s, c], axis=1)
    o_type[g] = g_type > 0.5
    o_role[g] = g_role > 0.5
    o_size[g] = g_size
    o_gcmd[g] = g_cmd

    H = _N_HIST
    t_row_h = jax.lax.broadcasted_iota(jnp.int32, (H, 2 * H), 1)
    t2_col_h = 2 * jax.lax.broadcasted_iota(jnp.int32, (H, 2 * H), 0)
    e0h = (t_row_h == t2_col_h).astype(jnp.float32)
    e1h = (t_row_h == t2_col_h + 1).astype(jnp.float32)
    F = T - H
    t_row_f = jax.lax.broadcasted_iota(jnp.int32, (F, 2 * F), 1)
    t2_col_f = 2 * jax.lax.broadcasted_iota(jnp.int32, (F, 2 * F), 0)
    e0f = (t_row_f == t2_col_f).astype(jnp.float32)
    e1f = (t_row_f == t2_col_f + 1).astype(jnp.float32)

    o_tvalid[g] = g_valid[:, :H] > 0.5
    o_tpos[g] = _dot(lx[:, :H], e0h) + _dot(ly[:, :H], e1h)
    o_tvel[g] = _dot(lvx[:, :H], e0h) + _dot(lvy[:, :H], e1h)
    o_tspd[g] = g_spd[:, :H]
    o_tacc[g] = g_acc[:, :H]
    o_tyaw[g] = lyaw[:, :H]
    o_tyawr[g] = g_yawr[:, :H]

    o_gvalid[g] = g_valid[:, H:] > 0.5
    o_gpos[g] = _dot(lx[:, H:], e0f) + _dot(ly[:, H:], e1f)
    o_gvel[g] = _dot(lvx[:, H:], e0f) + _dot(lvy[:, H:], e1f)
    o_gspd[g] = g_spd[:, H:]
    o_gyaw[g] = lyaw[:, H:]


def kernel(agent_valid, agent_pos, agent_vel, agent_spd, agent_acc,
           agent_yaw_bbox, agent_yaw_rate, agent_type, agent_role,
           agent_size, agent_cmd):
    S, T, A = agent_valid.shape
    P = _N_TARGET
    H = _N_HIST
    F = T - H

    pos = agent_pos.reshape(S, T, 2 * A)
    vel = agent_vel.reshape(S, T, 2 * A)
    spd = agent_spd.reshape(S, T, A)
    acc = agent_acc.reshape(S, T, A)
    yaw = agent_yaw_bbox.reshape(S, T, A)
    yawr = agent_yaw_rate.reshape(S, T, A)

    f32 = jnp.float32
    out_shapes = (
        jax.ShapeDtypeStruct((S, 1, P), jnp.int32),
        jax.ShapeDtypeStruct((S, P, 2), f32),
        jax.ShapeDtypeStruct((S, P, 4), f32),
        jax.ShapeDtypeStruct((S, P, 3), jnp.bool_),
        jax.ShapeDtypeStruct((S, P, 3), jnp.bool_),
        jax.ShapeDtypeStruct((S, P, H), jnp.bool_),
        jax.ShapeDtypeStruct((S, P, 2 * H), f32),
        jax.ShapeDtypeStruct((S, P, 2 * H), f32),
        jax.ShapeDtypeStruct((S, P, H), f32),
        jax.ShapeDtypeStruct((S, P, H), f32),
        jax.ShapeDtypeStruct((S, P, H), f32),
        jax.ShapeDtypeStruct((S, P, H), f32),
        jax.ShapeDtypeStruct((S, P, 3), f32),
        jax.ShapeDtypeStruct((S, P, F), jnp.bool_),
        jax.ShapeDtypeStruct((S, P, 2 * F), f32),
        jax.ShapeDtypeStruct((S, P, F), f32),
        jax.ShapeDtypeStruct((S, P, 2 * F), f32),
        jax.ShapeDtypeStruct((S, P, F), f32),
        jax.ShapeDtypeStruct((S, P, 8), f32),
    )

    G = _SCENES_PER_STEP

    def spec(*dims):
        return pl.BlockSpec((G,) + dims, lambda s: (s,) + (0,) * len(dims))

    outs = pl.pallas_call(
        _group_kernel,
        grid=(S // G,),
        compiler_params=pltpu.CompilerParams(
            dimension_semantics=("arbitrary",)),
        in_specs=[
            spec(T, A),
            spec(T, 2 * A),
            spec(T, 2 * A),
            spec(T, A),
            spec(T, A),
            spec(T, A),
            spec(T, A),
            spec(A, 3),
            spec(A, 3),
            spec(A, 3),
            spec(A, 8),
        ],
        out_specs=tuple(spec(*o.shape[1:]) for o in out_shapes),
        out_shape=out_shapes,
    )(agent_valid, pos, vel, spd, acc, yaw, yawr,
      agent_type, agent_role, agent_size, agent_cmd)

    (o_idx, o_refpos, o_refrot, o_type, o_role, o_tvalid, o_tpos, o_tvel,
     o_tspd, o_tacc, o_tyaw, o_tyawr, o_size, o_gvalid, o_gpos, o_gspd,
     o_gvel, o_gyaw, o_gcmd) = outs

    target_indices = o_idx.reshape(S, P)
    ref_pos = o_refpos.reshape(S, P, 1, 2)
    ref_rot = o_refrot.reshape(S, P, 2, 2)
    tgt_pos = o_tpos.reshape(S, P, H, 2)
    tgt_vel = o_tvel.reshape(S, P, H, 2)
    gt_pos = o_gpos.reshape(S, P, F, 2)
    gt_vel = o_gvel.reshape(S, P, F, 2)

    return (target_indices, ref_pos, ref_rot, o_type, o_role,
            o_tvalid, tgt_pos, tgt_vel,
            o_tspd.reshape(S, P, H, 1), o_tacc.reshape(S, P, H, 1),
            o_tyaw.reshape(S, P, H, 1), o_tyawr.reshape(S, P, H, 1),
            o_type, o_role, o_size,
            o_gvalid, gt_pos, o_gspd.reshape(S, P, F, 1), gt_vel,
            o_gyaw.reshape(S, P, F, 1), o_gcmd)


# speedup vs baseline: 1.2928x; 1.1458x over previous
import jax
import jax.numpy as jnp
from jax.experimental import pallas as pl
from jax.experimental.pallas import tpu as pltpu

_STEP_CURRENT = 10
_N_HIST = _STEP_CURRENT + 1
_N_TARGET = 8
_PI = 3.141592653589793
_HI = jax.lax.Precision.HIGHEST
_LO = jax.lax.Precision.DEFAULT
_SCENES_PER_STEP = 8


def _wrap_rad(x):
    m = x + _PI
    m = m - (2.0 * _PI) * jnp.floor(m / (2.0 * _PI))
    return m - _PI


def _dot_t(a, b, prec=_HI):
    return jax.lax.dot_general(
        a, b, (((1,), (1,)), ((), ())), precision=prec,
        preferred_element_type=jnp.float32)


def _dot(a, b, prec=_HI):
    return jax.lax.dot_general(
        a, b, (((1,), (0,)), ((), ())), precision=prec,
        preferred_element_type=jnp.float32)


def _group_kernel(valid_ref, pos_ref, vel_ref, spd_ref, acc_ref, yaw_ref,
                  yawr_ref, type_ref, role_ref, size_ref, cmd_ref, *out_refs):
    for g in range(_SCENES_PER_STEP):
        _one_scene(g, valid_ref, pos_ref, vel_ref, spd_ref, acc_ref, yaw_ref,
                   yawr_ref, type_ref, role_ref, size_ref, cmd_ref, *out_refs)


def _one_scene(g, valid_ref, pos_ref, vel_ref, spd_ref, acc_ref, yaw_ref,
               yawr_ref, type_ref, role_ref, size_ref, cmd_ref,
               o_idx, o_refpos, o_refrot, o_type, o_role,
               o_tvalid, o_tpos, o_tvel, o_tspd, o_tacc, o_tyaw, o_tyawr,
               o_size, o_gvalid, o_gpos, o_gspd, o_gvel, o_gyaw, o_gcmd):
    A = 64
    P = _N_TARGET
    T = valid_ref.shape[1]

    valid = valid_ref[g].astype(jnp.float32)
    role = role_ref[g].astype(jnp.float32)

    w_col = jnp.sum(role, axis=1, keepdims=True)
    w_row = jnp.transpose(w_col) + valid[_STEP_CURRENT:_STEP_CURRENT + 1, :]
    key_row = w_row.astype(jnp.int32) * A + (A - 1 - jax.lax.broadcasted_iota(
        jnp.int32, (1, A), 1))
    key_col = jnp.transpose(key_row)
    rank_col = jnp.sum((key_row > key_col).astype(jnp.int32), axis=1,
                       keepdims=True)
    rank_row = jnp.transpose(rank_col)
    p_col = jax.lax.broadcasted_iota(jnp.int32, (P, 1), 0)
    sel = (rank_row == p_col)
    a_row = jax.lax.broadcasted_iota(jnp.int32, (P, A), 1)
    idx_col = jnp.sum(jnp.where(sel, a_row, 0), axis=1, keepdims=True)
    o_idx[g] = jnp.transpose(idx_col)

    sel_f = sel.astype(jnp.float32)

    l_row = jax.lax.broadcasted_iota(jnp.int32, (A, 2 * A), 1)
    a2_col = 2 * jax.lax.broadcasted_iota(jnp.int32, (A, 2 * A), 0)
    d0 = (l_row == a2_col).astype(jnp.float32)
    d1 = (l_row == a2_col + 1).astype(jnp.float32)
    sel2x = _dot(sel_f, d0, _LO)
    sel2y = _dot(sel_f, d1, _LO)

    pos = pos_ref[g]
    vel = vel_ref[g]
    px = _dot_t(sel2x, pos)
    py = _dot_t(sel2y, pos)
    vx = _dot_t(sel2x, vel)
    vy = _dot_t(sel2y, vel)
    g_spd = _dot_t(sel_f, spd_ref[g], _LO)
    g_acc = _dot_t(sel_f, acc_ref[g], _LO)
    g_yaw = _dot_t(sel_f, yaw_ref[g])
    g_yawr = _dot_t(sel_f, yawr_ref[g], _LO)
    g_valid = _dot_t(sel_f, valid, _LO)

    g_type = _dot(sel_f, type_ref[g].astype(jnp.float32), _LO)
    g_role = _dot(sel_f, role, _LO)
    g_size = _dot(sel_f, size_ref[g], _LO)
    g_cmd = _dot(sel_f, cmd_ref[g], _LO)

    px0 = px[:, _STEP_CURRENT:_STEP_CURRENT + 1]
    py0 = py[:, _STEP_CURRENT:_STEP_CURRENT + 1]
    yaw0 = g_yaw[:, _STEP_CURRENT:_STEP_CURRENT + 1]
    c = jnp.cos(yaw0)
    s = jnp.sin(yaw0)

    dx = px - px0
    dy = py - py0
    lx = dx * c + dy * s
    ly = dy * c - dx * s
    lvx = vx * c + vy * s
    lvy = vy * c - vx * s
    lyaw = _wrap_rad(g_yaw - yaw0)

    o_refpos[g] = jnp.concatenate([px0, py0], axis=1)
    o_refrot[g] = jnp.concatenate([c, -s, s, c], axis=1)
    o_type[g] = g_type > 0.5
    o_role[g] = g_role > 0.5
    o_size[g] = g_size
    o_gcmd[g] = g_cmd

    H = _N_HIST
    t_row_h = jax.lax.broadcasted_iota(jnp.int32, (H, 2 * H), 1)
    t2_col_h = 2 * jax.lax.broadcasted_iota(jnp.int32, (H, 2 * H), 0)
    e0h = (t_row_h == t2_col_h).astype(jnp.float32)
    e1h = (t_row_h == t2_col_h + 1).astype(jnp.float32)
    F = T - H
    t_row_f = jax.lax.broadcasted_iota(jnp.int32, (F, 2 * F), 1)
    t2_col_f = 2 * jax.lax.broadcasted_iota(jnp.int32, (F, 2 * F), 0)
    e0f = (t_row_f == t2_col_f).astype(jnp.float32)
    e1f = (t_row_f == t2_col_f + 1).astype(jnp.float32)

    o_tvalid[g] = g_valid[:, :H] > 0.5
    o_tpos[g] = _dot(lx[:, :H], e0h, _LO) + _dot(ly[:, :H], e1h, _LO)
    o_tvel[g] = _dot(lvx[:, :H], e0h, _LO) + _dot(lvy[:, :H], e1h, _LO)
    o_tspd[g] = g_spd[:, :H]
    o_tacc[g] = g_acc[:, :H]
    o_tyaw[g] = lyaw[:, :H]
    o_tyawr[g] = g_yawr[:, :H]

    o_gvalid[g] = g_valid[:, H:] > 0.5
    o_gpos[g] = _dot(lx[:, H:], e0f, _LO) + _dot(ly[:, H:], e1f, _LO)
    o_gvel[g] = _dot(lvx[:, H:], e0f, _LO) + _dot(lvy[:, H:], e1f, _LO)
    o_gspd[g] = g_spd[:, H:]
    o_gyaw[g] = lyaw[:, H:]


def kernel(agent_valid, agent_pos, agent_vel, agent_spd, agent_acc,
           agent_yaw_bbox, agent_yaw_rate, agent_type, agent_role,
           agent_size, agent_cmd):
    S, T, A = agent_valid.shape
    P = _N_TARGET
    H = _N_HIST
    F = T - H

    pos = agent_pos.reshape(S, T, 2 * A)
    vel = agent_vel.reshape(S, T, 2 * A)
    spd = agent_spd.reshape(S, T, A)
    acc = agent_acc.reshape(S, T, A)
    yaw = agent_yaw_bbox.reshape(S, T, A)
    yawr = agent_yaw_rate.reshape(S, T, A)

    f32 = jnp.float32
    out_shapes = (
        jax.ShapeDtypeStruct((S, 1, P), jnp.int32),
        jax.ShapeDtypeStruct((S, P, 2), f32),
        jax.ShapeDtypeStruct((S, P, 4), f32),
        jax.ShapeDtypeStruct((S, P, 3), jnp.bool_),
        jax.ShapeDtypeStruct((S, P, 3), jnp.bool_),
        jax.ShapeDtypeStruct((S, P, H), jnp.bool_),
        jax.ShapeDtypeStruct((S, P, 2 * H), f32),
        jax.ShapeDtypeStruct((S, P, 2 * H), f32),
        jax.ShapeDtypeStruct((S, P, H), f32),
        jax.ShapeDtypeStruct((S, P, H), f32),
        jax.ShapeDtypeStruct((S, P, H), f32),
        jax.ShapeDtypeStruct((S, P, H), f32),
        jax.ShapeDtypeStruct((S, P, 3), f32),
        jax.ShapeDtypeStruct((S, P, F), jnp.bool_),
        jax.ShapeDtypeStruct((S, P, 2 * F), f32),
        jax.ShapeDtypeStruct((S, P, F), f32),
        jax.ShapeDtypeStruct((S, P, 2 * F), f32),
        jax.ShapeDtypeStruct((S, P, F), f32),
        jax.ShapeDtypeStruct((S, P, 8), f32),
    )

    G = _SCENES_PER_STEP

    def spec(*dims):
        return pl.BlockSpec((G,) + dims, lambda s: (s,) + (0,) * len(dims))

    outs = pl.pallas_call(
        _group_kernel,
        grid=(S // G,),
        compiler_params=pltpu.CompilerParams(
            dimension_semantics=("arbitrary",)),
        in_specs=[
            spec(T, A),
            spec(T, 2 * A),
            spec(T, 2 * A),
            spec(T, A),
            spec(T, A),
            spec(T, A),
            spec(T, A),
            spec(A, 3),
            spec(A, 3),
            spec(A, 3),
            spec(A, 8),
        ],
        out_specs=tuple(spec(*o.shape[1:]) for o in out_shapes),
        out_shape=out_shapes,
    )(agent_valid, pos, vel, spd, acc, yaw, yawr,
      agent_type, agent_role, agent_size, agent_cmd)

    (o_idx, o_refpos, o_refrot, o_type, o_role, o_tvalid, o_tpos, o_tvel,
     o_tspd, o_tacc, o_tyaw, o_tyawr, o_size, o_gvalid, o_gpos, o_gspd,
     o_gvel, o_gyaw, o_gcmd) = outs

    target_indices = o_idx.reshape(S, P)
    ref_pos = o_refpos.reshape(S, P, 1, 2)
    ref_rot = o_refrot.reshape(S, P, 2, 2)
    tgt_pos = o_tpos.reshape(S, P, H, 2)
    tgt_vel = o_tvel.reshape(S, P, H, 2)
    gt_pos = o_gpos.reshape(S, P, F, 2)
    gt_vel = o_gvel.reshape(S, P, F, 2)

    return (target_indices, ref_pos, ref_rot, o_type, o_role,
            o_tvalid, tgt_pos, tgt_vel,
            o_tspd.reshape(S, P, H, 1), o_tacc.reshape(S, P, H, 1),
            o_tyaw.reshape(S, P, H, 1), o_tyawr.reshape(S, P, H, 1),
            o_type, o_role, o_size,
            o_gvalid, gt_pos, o_gspd.reshape(S, P, F, 1), gt_vel,
            o_gyaw.reshape(S, P, F, 1), o_gcmd)
